# Initial kernel scaffold; baseline (speedup 1.0000x reference)
import jax
import jax.numpy as jnp
from jax.experimental import pallas as pl


def kernel(label, proto):
    raise NotImplementedError("write your pallas kernel here")


# speedup vs baseline: 1.4653x; 1.4653x over previous
import functools

import jax
import jax.numpy as jnp
from jax import lax
from jax.experimental import pallas as pl
from jax.experimental.pallas import tpu as pltpu
from jax.experimental.pallas import tpu_sc as plsc

FEAT = 1024
NCLS = 1000
VPAD = 1024
BATCH = 16384

NC, NS = 2, 16
NW = NC * NS
BPW = BATCH // NW
CHUNK = 32
NCHUNK = BPW // CHUNK


def _transpose_body(p_ref, o_ref):
    o_ref[...] = p_ref[...].T


def _transpose(p_pad):
    return pl.pallas_call(
        _transpose_body,
        out_shape=jax.ShapeDtypeStruct((VPAD, FEAT), jnp.float32),
    )(p_pad)


def _gather_body(table_hbm, idx_hbm, out_hbm, idx_v, rows_v, gsem0, gsem1,
                 ssem0, ssem1):
    gsems = (gsem0, gsem1)
    ssems = (ssem0, ssem1)
    wid = lax.axis_index("s") * NC + lax.axis_index("c")
    base = wid * BPW
    pltpu.sync_copy(idx_hbm.at[pl.ds(base, BPW)], idx_v)

    def gather_start(g, b):
        pltpu.make_async_copy(
            table_hbm.at[idx_v.at[pl.ds(g * CHUNK, CHUNK)]],
            rows_v.at[b],
            gsems[b],
        ).start()

    for b in range(2):
        gather_start(b, b)

    def body(j, _):
        for b in range(2):
            g = 2 * j + b
            pltpu.make_async_copy(
                table_hbm.at[idx_v.at[pl.ds(g * CHUNK, CHUNK)]],
                rows_v.at[b],
                gsems[b],
            ).wait()
            out_slice = out_hbm.at[pl.ds(base + g * CHUNK, CHUNK)]
            pltpu.make_async_copy(rows_v.at[b], out_slice, ssems[b]).start()
            pltpu.make_async_copy(rows_v.at[b], out_slice, ssems[b]).wait()

            @pl.when(g + 2 < NCHUNK)
            def _():
                gather_start(g + 2, b)
        return 0

    lax.fori_loop(0, NCHUNK // 2, body, 0)


def _sc_gather(tableT, label):
    mesh = plsc.VectorSubcoreMesh(core_axis_name="c", subcore_axis_name="s")
    return pl.kernel(
        _gather_body,
        out_type=jax.ShapeDtypeStruct((BATCH, FEAT), jnp.float32),
        mesh=mesh,
        scratch_types=[
            pltpu.VMEM((BPW,), jnp.int32),
            pltpu.VMEM((2, CHUNK, FEAT), jnp.float32),
            pltpu.SemaphoreType.DMA,
            pltpu.SemaphoreType.DMA,
            pltpu.SemaphoreType.DMA,
            pltpu.SemaphoreType.DMA,
        ],
    )(tableT, label)


def kernel(label, proto):
    p_pad = jnp.pad(proto, ((0, 0), (0, VPAD - NCLS)))
    tableT = _transpose(p_pad)
    return _sc_gather(tableT, label.astype(jnp.int32))


# speedup vs baseline: 1.4765x; 1.0077x over previous
import functools

import jax
import jax.numpy as jnp
from jax import lax
from jax.experimental import pallas as pl
from jax.experimental.pallas import tpu as pltpu
from jax.experimental.pallas import tpu_sc as plsc

FEAT = 1024
NCLS = 1000
VPAD = 1024
BATCH = 16384

NC, NS = 2, 16
NW = NC * NS
BPW = BATCH // NW
CHUNK = 32
NCHUNK = BPW // CHUNK


def _transpose_body(p_ref, o_ref):
    o_ref[0:NCLS, :] = p_ref[...].T


def _transpose(proto):
    return pl.pallas_call(
        _transpose_body,
        out_shape=jax.ShapeDtypeStruct((VPAD, FEAT), jnp.float32),
    )(proto)


def _gather_body(table_hbm, idx_hbm, out_hbm, idx_v, rows_v, gsem0, gsem1,
                 ssem0, ssem1):
    gsems = (gsem0, gsem1)
    ssems = (ssem0, ssem1)
    wid = lax.axis_index("s") * NC + lax.axis_index("c")
    base = wid * BPW
    pltpu.sync_copy(idx_hbm.at[pl.ds(base, BPW)], idx_v)

    def gather_start(g, b):
        pltpu.make_async_copy(
            table_hbm.at[idx_v.at[pl.ds(g * CHUNK, CHUNK)]],
            rows_v.at[b],
            gsems[b],
        ).start()

    for b in range(2):
        gather_start(b, b)

    def body(j, _):
        for b in range(2):
            g = 2 * j + b
            pltpu.make_async_copy(
                table_hbm.at[idx_v.at[pl.ds(g * CHUNK, CHUNK)]],
                rows_v.at[b],
                gsems[b],
            ).wait()
            out_slice = out_hbm.at[pl.ds(base + g * CHUNK, CHUNK)]
            pltpu.make_async_copy(rows_v.at[b], out_slice, ssems[b]).start()
            pltpu.make_async_copy(rows_v.at[b], out_slice, ssems[b]).wait()

            @pl.when(g + 2 < NCHUNK)
            def _():
                gather_start(g + 2, b)
        return 0

    lax.fori_loop(0, NCHUNK // 2, body, 0)


def _sc_gather(tableT, label):
    mesh = plsc.VectorSubcoreMesh(core_axis_name="c", subcore_axis_name="s")
    return pl.kernel(
        _gather_body,
        out_type=jax.ShapeDtypeStruct((BATCH, FEAT), jnp.float32),
        mesh=mesh,
        scratch_types=[
            pltpu.VMEM((BPW,), jnp.int32),
            pltpu.VMEM((2, CHUNK, FEAT), jnp.float32),
            pltpu.SemaphoreType.DMA,
            pltpu.SemaphoreType.DMA,
            pltpu.SemaphoreType.DMA,
            pltpu.SemaphoreType.DMA,
        ],
    )(tableT, label)


def kernel(label, proto):
    tableT = _transpose(proto)
    return _sc_gather(tableT, label.astype(jnp.int32))


# speedup vs baseline: 1.4818x; 1.0035x over previous
import functools

import jax
import jax.numpy as jnp
from jax import lax
from jax.experimental import pallas as pl
from jax.experimental.pallas import tpu as pltpu
from jax.experimental.pallas import tpu_sc as plsc

FEAT = 1024
NCLS = 1000
VPAD = 1024
BATCH = 16384

NC, NS = 2, 16
NW = NC * NS
BPW = BATCH // NW
CHUNK = 16
NCHUNK = BPW // CHUNK
NBUF = 4


def _transpose_body(p_ref, o_ref):
    o_ref[0:NCLS, :] = p_ref[...].T


def _transpose(proto):
    return pl.pallas_call(
        _transpose_body,
        out_shape=jax.ShapeDtypeStruct((VPAD, FEAT), jnp.float32),
    )(proto)


def _gather_body(table_hbm, idx_hbm, out_hbm, idx_v, rows_v, gsem0, gsem1,
                 gsem2, gsem3, ssem0, ssem1, ssem2, ssem3):
    gsems = (gsem0, gsem1, gsem2, gsem3)
    ssems = (ssem0, ssem1, ssem2, ssem3)
    wid = lax.axis_index("s") * NC + lax.axis_index("c")
    base = wid * BPW
    pltpu.sync_copy(idx_hbm.at[pl.ds(base, BPW)], idx_v)

    def gather(g, b):
        return pltpu.make_async_copy(
            table_hbm.at[idx_v.at[pl.ds(g * CHUNK, CHUNK)]],
            rows_v.at[b],
            gsems[b],
        )

    def scatter(g, b):
        return pltpu.make_async_copy(
            rows_v.at[b],
            out_hbm.at[pl.ds(base + g * CHUNK, CHUNK)],
            ssems[b],
        )

    for b in range(NBUF - 1):
        gather(b, b).start()

    def body(j, _):
        for b in range(NBUF):
            g = NBUF * j + b

            @pl.when(g >= 1)
            def _():
                scatter(g - 1, (b - 1) % NBUF).wait()

            @pl.when(g + NBUF - 1 < NCHUNK)
            def _():
                gather(g + NBUF - 1, (b + NBUF - 1) % NBUF).start()

            gather(g, b).wait()
            scatter(g, b).start()
        return 0

    lax.fori_loop(0, NCHUNK // NBUF, body, 0)
    scatter(NCHUNK - 1, (NCHUNK - 1) % NBUF).wait()


def _sc_gather(tableT, label):
    mesh = plsc.VectorSubcoreMesh(core_axis_name="c", subcore_axis_name="s")
    return pl.kernel(
        _gather_body,
        out_type=jax.ShapeDtypeStruct((BATCH, FEAT), jnp.float32),
        mesh=mesh,
        scratch_types=[
            pltpu.VMEM((BPW,), jnp.int32),
            pltpu.VMEM((NBUF, CHUNK, FEAT), jnp.float32),
        ] + [pltpu.SemaphoreType.DMA] * (2 * NBUF),
    )(tableT, label)


def kernel(label, proto):
    tableT = _transpose(proto)
    return _sc_gather(tableT, label.astype(jnp.int32))
